# Initial kernel scaffold; baseline (speedup 1.0000x reference)
import jax
import jax.numpy as jnp
from jax.experimental import pallas as pl


def kernel(logits, labels):
    raise NotImplementedError("write your pallas kernel here")


# speedup vs baseline: 1.9176x; 1.9176x over previous
import functools

import jax
import jax.numpy as jnp
from jax import lax
from jax.experimental import pallas as pl
from jax.experimental.pallas import tpu as pltpu
from jax.experimental.pallas import tpu_sc as plsc

_NC = 2
_NS = 16
_NW = _NC * _NS
_L = 16
_NBINS = 10
_HIST = 256


def _sc_hist_kernel(n_total):
    chunk = n_total // _NW
    nvec = chunk // _L
    mesh = plsc.VectorSubcoreMesh(core_axis_name="c", subcore_axis_name="s")
    out_t = [jax.ShapeDtypeStruct((_NW, _L), jnp.float32)] * 3

    @functools.partial(
        pl.kernel,
        out_type=out_t,
        mesh=mesh,
        compiler_params=pltpu.CompilerParams(needs_layout_passes=False),
        scratch_types=[
            pltpu.VMEM((2 * chunk,), jnp.float32),
            pltpu.VMEM((chunk,), jnp.int32),
            pltpu.VMEM((_HIST,), jnp.float32),
            pltpu.VMEM((_HIST,), jnp.float32),
            pltpu.VMEM((_HIST,), jnp.float32),
            pltpu.VMEM((_L,), jnp.float32),
            pltpu.VMEM((_L,), jnp.float32),
            pltpu.VMEM((_L,), jnp.float32),
        ],
    )
    def body(logits_hbm, labels_hbm, cnt_out, lab_out, prd_out,
             log_v, lab_v, cnt_h, lab_h, prd_h, s_cnt, s_lab, s_prd):
        wid = lax.axis_index("c") * _NS + lax.axis_index("s")

        pltpu.sync_copy(logits_hbm.at[pl.ds(wid * 2 * chunk, 2 * chunk)], log_v)
        pltpu.sync_copy(labels_hbm.at[pl.ds(wid * chunk, chunk)], lab_v)

        zeros = jnp.zeros((_L,), jnp.float32)
        for j in range(_HIST // _L):
            cnt_h[pl.ds(j * _L, _L)] = zeros
            lab_h[pl.ds(j * _L, _L)] = zeros
            prd_h[pl.ds(j * _L, _L)] = zeros

        lane = lax.iota(jnp.int32, _L)
        two_lane = lane * 2
        ones = jnp.ones((_L,), jnp.float32)

        def step(i, carry):
            idx0 = two_lane + i * (2 * _L)
            g0 = plsc.load_gather(log_v, [idx0])
            g1 = plsc.load_gather(log_v, [idx0 + 1])
            d = g1 - g0
            t = jnp.exp(-jnp.abs(d))
            r = 1.0 / (1.0 + t)
            p = jnp.where(d >= 0, r, t * r)
            b = (p * 10.0).astype(jnp.int32)
            sidx = b * _L + lane
            labf = lab_v[pl.ds(i * _L, _L)].astype(jnp.float32)
            plsc.addupdate_scatter(cnt_h, [sidx], ones)
            plsc.addupdate_scatter(lab_h, [sidx], labf)
            plsc.addupdate_scatter(prd_h, [sidx], p)
            return carry

        lax.fori_loop(0, nvec, step, 0)

        for h, st, out in ((cnt_h, s_cnt, cnt_out),
                           (lab_h, s_lab, lab_out),
                           (prd_h, s_prd, prd_out)):
            acc = zeros
            for l in range(_L):
                acc = acc + plsc.load_gather(h, [lane * _L + l])
            st[...] = acc
            pltpu.sync_copy(st, out.at[wid])

    return body


def _tc_finish(cnt_ref, lab_ref, prd_ref, out_ref):
    cnt = jnp.sum(cnt_ref[...], axis=0, keepdims=True)
    lab = jnp.sum(lab_ref[...], axis=0, keepdims=True)
    prd = jnp.sum(prd_ref[...], axis=0, keepdims=True)
    mask = lax.broadcasted_iota(jnp.int32, (1, _L), 1) < _NBINS
    nz = jnp.logical_and(cnt > 0, mask)
    safe = jnp.where(nz, cnt, 1.0)
    accs = jnp.where(nz, lab / safe, 0.0)
    confs = jnp.where(nz, prd / safe, 0.0)
    diff = jnp.abs(accs - confs)
    cntm = jnp.where(mask, cnt, 0.0)
    total = jnp.sum(cntm, axis=1, keepdims=True)
    out_ref[...] = jnp.sum(cntm * diff, axis=1, keepdims=True) / total


def kernel(logits, labels):
    n = logits.shape[0]
    cnt, lab, prd = _sc_hist_kernel(n)(logits.reshape(-1), labels)
    ece = pl.pallas_call(
        _tc_finish,
        out_shape=jax.ShapeDtypeStruct((1, 1), jnp.float32),
    )(cnt, lab, prd)
    return ece[0, 0]


# speedup vs baseline: 31.2387x; 16.2904x over previous
import functools

import jax
import jax.numpy as jnp
from jax import lax
from jax.experimental import pallas as pl
from jax.experimental.pallas import tpu as pltpu
from jax.experimental.pallas import tpu_sc as plsc

_NC = 2
_NS = 16
_NW = _NC * _NS
_L = 16
_NBINS = 10
_HIST = 256


def _sc_hist_kernel(n_total):
    chunk = n_total // _NW
    crows = chunk // 128
    mesh = plsc.VectorSubcoreMesh(core_axis_name="c", subcore_axis_name="s")
    out_t = [jax.ShapeDtypeStruct((_NW, _L), jnp.float32)] * 3

    @functools.partial(
        pl.kernel,
        out_type=out_t,
        mesh=mesh,
        compiler_params=pltpu.CompilerParams(needs_layout_passes=False),
        scratch_types=[
            pltpu.VMEM((2 * crows, 128), jnp.float32),
            pltpu.VMEM((crows, 128), jnp.int32),
            pltpu.VMEM((_HIST,), jnp.float32),
            pltpu.VMEM((_HIST,), jnp.float32),
            pltpu.VMEM((_HIST,), jnp.float32),
            pltpu.VMEM((_L,), jnp.float32),
            pltpu.VMEM((_L,), jnp.float32),
            pltpu.VMEM((_L,), jnp.float32),
        ],
    )
    def body(logits_hbm, labels_hbm, cnt_out, lab_out, prd_out,
             log_v, lab_v, cnt_h, lab_h, prd_h, s_cnt, s_lab, s_prd):
        wid = lax.axis_index("c") * _NS + lax.axis_index("s")

        pltpu.sync_copy(logits_hbm.at[pl.ds(wid * 2 * crows, 2 * crows)], log_v)
        pltpu.sync_copy(labels_hbm.at[pl.ds(wid * crows, crows)], lab_v)

        zeros = jnp.zeros((_L,), jnp.float32)
        for j in range(_HIST // _L):
            cnt_h[pl.ds(j * _L, _L)] = zeros
            lab_h[pl.ds(j * _L, _L)] = zeros
            prd_h[pl.ds(j * _L, _L)] = zeros

        lane = lax.iota(jnp.int32, _L)
        ones = jnp.ones((_L,), jnp.float32)

        def step(r, carry):
            for v in range(128 // _L):
                g0 = log_v[2 * r, pl.ds(v * _L, _L)]
                g1 = log_v[2 * r + 1, pl.ds(v * _L, _L)]
                labf = lab_v[r, pl.ds(v * _L, _L)].astype(jnp.float32)
                d = g1 - g0
                t = jnp.exp(-jnp.abs(d))
                rec = 1.0 / (1.0 + t)
                p = jnp.where(d >= 0, rec, t * rec)
                b = (p * 10.0).astype(jnp.int32)
                sidx = b * _L + lane
                plsc.addupdate_scatter(cnt_h, [sidx], ones)
                plsc.addupdate_scatter(lab_h, [sidx], labf)
                plsc.addupdate_scatter(prd_h, [sidx], p)
            return carry

        lax.fori_loop(0, crows, step, 0)

        for h, st, out in ((cnt_h, s_cnt, cnt_out),
                           (lab_h, s_lab, lab_out),
                           (prd_h, s_prd, prd_out)):
            acc = zeros
            for l in range(_L):
                acc = acc + plsc.load_gather(h, [lane * _L + l])
            st[...] = acc
            pltpu.sync_copy(st, out.at[wid])

    return body


def _tc_finish(cnt_ref, lab_ref, prd_ref, out_ref):
    cnt = jnp.sum(cnt_ref[...], axis=0, keepdims=True)
    lab = jnp.sum(lab_ref[...], axis=0, keepdims=True)
    prd = jnp.sum(prd_ref[...], axis=0, keepdims=True)
    mask = lax.broadcasted_iota(jnp.int32, (1, _L), 1) < _NBINS
    nz = jnp.logical_and(cnt > 0, mask)
    safe = jnp.where(nz, cnt, 1.0)
    accs = jnp.where(nz, lab / safe, 0.0)
    confs = jnp.where(nz, prd / safe, 0.0)
    diff = jnp.abs(accs - confs)
    cntm = jnp.where(mask, cnt, 0.0)
    total = jnp.sum(cntm, axis=1, keepdims=True)
    out_ref[...] = jnp.sum(cntm * diff, axis=1, keepdims=True) / total


def kernel(logits, labels):
    n = logits.shape[0]
    logits2d = logits.reshape(n // 128, 128, 2).transpose(0, 2, 1)
    logits2d = logits2d.reshape(2 * n // 128, 128)
    labels2d = labels.reshape(n // 128, 128)
    cnt, lab, prd = _sc_hist_kernel(n)(logits2d, labels2d)
    ece = pl.pallas_call(
        _tc_finish,
        out_shape=jax.ShapeDtypeStruct((1, 1), jnp.float32),
    )(cnt, lab, prd)
    return ece[0, 0]


# speedup vs baseline: 63.5452x; 2.0342x over previous
import functools

import jax
import jax.numpy as jnp
from jax import lax
from jax.experimental import pallas as pl
from jax.experimental.pallas import tpu as pltpu
from jax.experimental.pallas import tpu_sc as plsc

_NC = 2
_NS = 16
_NW = _NC * _NS
_L = 16
_NBINS = 10
_K = 4
_HIST = 256


def _sc_hist_kernel(n_total):
    chunk = n_total // _NW
    crows = chunk // 128
    mesh = plsc.VectorSubcoreMesh(core_axis_name="c", subcore_axis_name="s")
    out_t = [jax.ShapeDtypeStruct((_NW, _L), jnp.float32)] * 3

    @functools.partial(
        pl.kernel,
        out_type=out_t,
        mesh=mesh,
        compiler_params=pltpu.CompilerParams(needs_layout_passes=False),
        scratch_types=[
            pltpu.VMEM((2 * crows, 128), jnp.float32),
            pltpu.VMEM((crows, 128), jnp.int32),
            [pltpu.VMEM((_HIST,), jnp.int32) for _ in range(_K)],
            [pltpu.VMEM((_HIST,), jnp.float32) for _ in range(_K)],
            pltpu.VMEM((_L,), jnp.float32),
            pltpu.VMEM((_L,), jnp.float32),
            pltpu.VMEM((_L,), jnp.float32),
        ],
    )
    def body(logits_hbm, labels_hbm, cnt_out, lab_out, prd_out,
             log_v, lab_v, ch, ph, s_cnt, s_lab, s_prd):
        wid = lax.axis_index("c") * _NS + lax.axis_index("s")

        pltpu.sync_copy(logits_hbm.at[pl.ds(wid * 2 * crows, 2 * crows)], log_v)
        pltpu.sync_copy(labels_hbm.at[pl.ds(wid * crows, crows)], lab_v)

        zf = jnp.zeros((_L,), jnp.float32)
        zi = jnp.zeros((_L,), jnp.int32)
        for j in range(_HIST // _L):
            for k in range(_K):
                ch[k][pl.ds(j * _L, _L)] = zi
                ph[k][pl.ds(j * _L, _L)] = zf

        lane = lax.iota(jnp.int32, _L)

        def step(r, carry):
            for g in range(128 // _L // _K):
                base = g * _K
                g0 = [log_v[2 * r, pl.ds((base + k) * _L, _L)] for k in range(_K)]
                g1 = [log_v[2 * r + 1, pl.ds((base + k) * _L, _L)] for k in range(_K)]
                lb = [lab_v[r, pl.ds((base + k) * _L, _L)] for k in range(_K)]
                t = [jnp.exp(g0[k] - g1[k]) for k in range(_K)]
                p = [1.0 / (1.0 + t[k]) for k in range(_K)]
                ix = [(p[k] * 10.0).astype(jnp.int32) * _L + lane for k in range(_K)]
                cb = [lb[k] + 65536 for k in range(_K)]
                for k in range(_K):
                    plsc.addupdate_scatter(ch[k], [ix[k]], cb[k])
                for k in range(_K):
                    plsc.addupdate_scatter(ph[k], [ix[k]], p[k])
            return carry

        lax.fori_loop(0, crows, step, 0)

        lidx = lane * _L
        cnt_acc = zi
        lab_acc = zi
        prd_acc = zf
        for k in range(_K):
            for l in range(_L):
                u = plsc.load_gather(ch[k], [lidx + l])
                cnt_acc = cnt_acc + lax.shift_right_logical(u, 16)
                lab_acc = lab_acc + lax.bitwise_and(u, 65535)
                prd_acc = prd_acc + plsc.load_gather(ph[k], [lidx + l])
        s_cnt[...] = cnt_acc.astype(jnp.float32)
        s_lab[...] = lab_acc.astype(jnp.float32)
        s_prd[...] = prd_acc
        pltpu.sync_copy(s_cnt, cnt_out.at[wid])
        pltpu.sync_copy(s_lab, lab_out.at[wid])
        pltpu.sync_copy(s_prd, prd_out.at[wid])

    return body


def _tc_finish(cnt_ref, lab_ref, prd_ref, out_ref):
    cnt = jnp.sum(cnt_ref[...], axis=0, keepdims=True)
    lab = jnp.sum(lab_ref[...], axis=0, keepdims=True)
    prd = jnp.sum(prd_ref[...], axis=0, keepdims=True)
    mask = lax.broadcasted_iota(jnp.int32, (1, _L), 1) < _NBINS
    nz = jnp.logical_and(cnt > 0, mask)
    safe = jnp.where(nz, cnt, 1.0)
    accs = jnp.where(nz, lab / safe, 0.0)
    confs = jnp.where(nz, prd / safe, 0.0)
    diff = jnp.abs(accs - confs)
    cntm = jnp.where(mask, cnt, 0.0)
    total = jnp.sum(cntm, axis=1, keepdims=True)
    out_ref[...] = jnp.sum(cntm * diff, axis=1, keepdims=True) / total


def kernel(logits, labels):
    n = logits.shape[0]
    logits2d = logits.reshape(n // 128, 128, 2).transpose(0, 2, 1)
    logits2d = logits2d.reshape(2 * n // 128, 128)
    labels2d = labels.reshape(n // 128, 128)
    cnt, lab, prd = _sc_hist_kernel(n)(logits2d, labels2d)
    ece = pl.pallas_call(
        _tc_finish,
        out_shape=jax.ShapeDtypeStruct((1, 1), jnp.float32),
    )(cnt, lab, prd)
    return ece[0, 0]


# speedup vs baseline: 71.6759x; 1.1280x over previous
import functools

import jax
import jax.numpy as jnp
from jax import lax
from jax.experimental import pallas as pl
from jax.experimental.pallas import tpu as pltpu
from jax.experimental.pallas import tpu_sc as plsc

_NC = 2
_NS = 16
_NW = _NC * _NS
_L = 16
_NBINS = 10
_K = 8
_HIST = 256


def _sc_hist_kernel(n_total):
    chunk = n_total // _NW
    crows = chunk // 128
    mesh = plsc.VectorSubcoreMesh(core_axis_name="c", subcore_axis_name="s")
    out_t = [jax.ShapeDtypeStruct((_NW, _L), jnp.float32)] * 3

    @functools.partial(
        pl.kernel,
        out_type=out_t,
        mesh=mesh,
        compiler_params=pltpu.CompilerParams(needs_layout_passes=False),
        scratch_types=[
            pltpu.VMEM((2 * crows, 128), jnp.float32),
            pltpu.VMEM((crows, 128), jnp.int32),
            [pltpu.VMEM((_HIST,), jnp.int32) for _ in range(_K)],
            [pltpu.VMEM((_HIST,), jnp.float32) for _ in range(_K)],
            pltpu.VMEM((_L,), jnp.float32),
            pltpu.VMEM((_L,), jnp.float32),
            pltpu.VMEM((_L,), jnp.float32),
        ],
    )
    def body(logits_hbm, labels_hbm, cnt_out, lab_out, prd_out,
             log_v, lab_v, ch, ph, s_cnt, s_lab, s_prd):
        wid = lax.axis_index("c") * _NS + lax.axis_index("s")

        pltpu.sync_copy(logits_hbm.at[pl.ds(wid * 2 * crows, 2 * crows)], log_v)
        pltpu.sync_copy(labels_hbm.at[pl.ds(wid * crows, crows)], lab_v)

        zf = jnp.zeros((_L,), jnp.float32)
        zi = jnp.zeros((_L,), jnp.int32)
        for j in range(_HIST // _L):
            for k in range(_K):
                ch[k][pl.ds(j * _L, _L)] = zi
                ph[k][pl.ds(j * _L, _L)] = zf

        lane = lax.iota(jnp.int32, _L)

        def step(r, carry):
            for g in range(128 // _L // _K):
                base = g * _K
                g0 = [log_v[2 * r, pl.ds((base + k) * _L, _L)] for k in range(_K)]
                g1 = [log_v[2 * r + 1, pl.ds((base + k) * _L, _L)] for k in range(_K)]
                lb = [lab_v[r, pl.ds((base + k) * _L, _L)] for k in range(_K)]
                t = [jnp.exp(g0[k] - g1[k]) for k in range(_K)]
                p = [1.0 / (1.0 + t[k]) for k in range(_K)]
                ix = [(p[k] * 10.0).astype(jnp.int32) * _L + lane for k in range(_K)]
                cb = [lb[k] + 65536 for k in range(_K)]
                for k in range(_K):
                    plsc.addupdate_scatter(ch[k], [ix[k]], cb[k])
                for k in range(_K):
                    plsc.addupdate_scatter(ph[k], [ix[k]], p[k])
            return carry

        lax.fori_loop(0, crows, step, 0)

        lidx = lane * _L
        cnt_acc = zi
        lab_acc = zi
        prd_acc = zf
        for k in range(_K):
            for l in range(_L):
                u = plsc.load_gather(ch[k], [lidx + l])
                cnt_acc = cnt_acc + lax.shift_right_logical(u, 16)
                lab_acc = lab_acc + lax.bitwise_and(u, 65535)
                prd_acc = prd_acc + plsc.load_gather(ph[k], [lidx + l])
        s_cnt[...] = cnt_acc.astype(jnp.float32)
        s_lab[...] = lab_acc.astype(jnp.float32)
        s_prd[...] = prd_acc
        pltpu.sync_copy(s_cnt, cnt_out.at[wid])
        pltpu.sync_copy(s_lab, lab_out.at[wid])
        pltpu.sync_copy(s_prd, prd_out.at[wid])

    return body


def _tc_finish(cnt_ref, lab_ref, prd_ref, out_ref):
    cnt = jnp.sum(cnt_ref[...], axis=0, keepdims=True)
    lab = jnp.sum(lab_ref[...], axis=0, keepdims=True)
    prd = jnp.sum(prd_ref[...], axis=0, keepdims=True)
    mask = lax.broadcasted_iota(jnp.int32, (1, _L), 1) < _NBINS
    nz = jnp.logical_and(cnt > 0, mask)
    safe = jnp.where(nz, cnt, 1.0)
    accs = jnp.where(nz, lab / safe, 0.0)
    confs = jnp.where(nz, prd / safe, 0.0)
    diff = jnp.abs(accs - confs)
    cntm = jnp.where(mask, cnt, 0.0)
    total = jnp.sum(cntm, axis=1, keepdims=True)
    out_ref[...] = jnp.sum(cntm * diff, axis=1, keepdims=True) / total


def kernel(logits, labels):
    n = logits.shape[0]
    logits2d = logits.reshape(n // 128, 128, 2).transpose(0, 2, 1)
    logits2d = logits2d.reshape(2 * n // 128, 128)
    labels2d = labels.reshape(n // 128, 128)
    cnt, lab, prd = _sc_hist_kernel(n)(logits2d, labels2d)
    ece = pl.pallas_call(
        _tc_finish,
        out_shape=jax.ShapeDtypeStruct((1, 1), jnp.float32),
    )(cnt, lab, prd)
    return ece[0, 0]
